# Initial kernel scaffold; baseline (speedup 1.0000x reference)
import jax
import jax.numpy as jnp
from jax.experimental import pallas as pl


def kernel(x, edge_index, batch, W1, b1, W2, b2, lin1_W, lin1_b, bn1_g, bn1_b, lin2_W, lin2_b, bn2_g, bn2_b):
    raise NotImplementedError("write your pallas kernel here")


# speedup vs baseline: 7.5487x; 7.5487x over previous
import functools

import jax
import jax.numpy as jnp
from jax import lax
from jax.experimental import pallas as pl
from jax.experimental.pallas import tpu as pltpu
from jax.experimental.pallas import tpu_sc as plsc

N = 10000
D = 128
E = 320000
G = 64

BN = 1000
NB = N // BN

NC = 2
NS = 16
NW = NC * NS
EPT = E // NW
CH = 80
NCH = EPT // CH
ZR = 632
ZL = N - (NS - 1) * ZR


def _mm_body(x_ref, w_ref, o_ref):
    o_ref[...] = jnp.dot(x_ref[...], w_ref[...],
                         preferred_element_type=jnp.float32)


def _mm(x, w):
    return pl.pallas_call(
        _mm_body,
        grid=(NB,),
        in_specs=[pl.BlockSpec((BN, D), lambda i: (i, 0)),
                  pl.BlockSpec((D, D), lambda i: (0, 0))],
        out_specs=pl.BlockSpec((BN, D), lambda i: (i, 0)),
        out_shape=jax.ShapeDtypeStruct((N, D), jnp.float32),
    )(x, w)


def _layer2_body(a_ref, b_ref, w_ref, o_ref):
    h = jnp.maximum(a_ref[0] + a_ref[1] + b_ref[...], 0.0)
    o_ref[...] = jnp.dot(h, w_ref[...], preferred_element_type=jnp.float32)


def _layer2(a, b, w):
    return pl.pallas_call(
        _layer2_body,
        grid=(NB,),
        in_specs=[pl.BlockSpec((NC, BN, D), lambda i: (0, i, 0)),
                  pl.BlockSpec((1, D), lambda i: (0, 0)),
                  pl.BlockSpec((D, D), lambda i: (0, 0))],
        out_specs=pl.BlockSpec((BN, D), lambda i: (i, 0)),
        out_shape=jax.ShapeDtypeStruct((N, D), jnp.float32),
    )(a, b, w)


def _head_body(a_ref, b2_ref, bt_ref, l1w, l1b, g1, be1, l2w, l2b, g2, be2,
               o_ref, sums, cnts):
    i = pl.program_id(0)

    @pl.when(i == 0)
    def _():
        sums[...] = jnp.zeros_like(sums)
        cnts[...] = jnp.zeros_like(cnts)

    h = jnp.maximum(a_ref[0] + a_ref[1] + b2_ref[...], 0.0)
    bt = bt_ref[0]
    gid = lax.broadcasted_iota(jnp.int32, (G, BN), 0)
    ohT = (gid == bt).astype(jnp.float32)
    dn = (((1,), (0,)), ((), ()))
    sums[...] += lax.dot_general(ohT, h, dn,
                                 preferred_element_type=jnp.float32, precision=lax.Precision.HIGHEST)
    cnts[...] += lax.dot_general(ohT, jnp.ones((BN, D), jnp.float32), dn,
                                 preferred_element_type=jnp.float32, precision=lax.Precision.HIGHEST)

    @pl.when(i == NB - 1)
    def _():
        pooled = sums[...] / jnp.maximum(cnts[...], 1.0)
        dnT = (((1,), (1,)), ((), ()))
        z = lax.dot_general(pooled, l1w[...], dnT,
                            preferred_element_type=jnp.float32) + l1b[...]
        z = jnp.maximum(z, 0.0)
        mu = jnp.mean(z, axis=0, keepdims=True)
        var = jnp.mean((z - mu) ** 2, axis=0, keepdims=True)
        z = (z - mu) * lax.rsqrt(var + 1e-5) * g1[...] + be1[...]
        z = lax.dot_general(z, l2w[...], dnT,
                            preferred_element_type=jnp.float32) + l2b[...]
        z = jnp.maximum(z, 0.0)
        mu = jnp.mean(z, axis=0, keepdims=True)
        var = jnp.mean((z - mu) ** 2, axis=0, keepdims=True)
        z = (z - mu) * lax.rsqrt(var + 1e-5) * g2[...] + be2[...]
        mx = jnp.max(z, axis=1, keepdims=True)
        z = z - mx
        o_ref[...] = z - jnp.log(jnp.sum(jnp.exp(z), axis=1, keepdims=True))


def _head(a, b2, bt, l1w, l1b, g1, be1, l2w, l2b, g2, be2):
    H1 = l1w.shape[0]
    H2 = l2w.shape[0]
    return pl.pallas_call(
        _head_body,
        grid=(NB,),
        in_specs=[pl.BlockSpec((NC, BN, D), lambda i: (0, i, 0)),
                  pl.BlockSpec((1, D), lambda i: (0, 0)),
                  pl.BlockSpec((1, 1, BN), lambda i: (i, 0, 0)),
                  pl.BlockSpec((H1, D), lambda i: (0, 0)),
                  pl.BlockSpec((1, H1), lambda i: (0, 0)),
                  pl.BlockSpec((1, H1), lambda i: (0, 0)),
                  pl.BlockSpec((1, H1), lambda i: (0, 0)),
                  pl.BlockSpec((H2, H1), lambda i: (0, 0)),
                  pl.BlockSpec((1, H2), lambda i: (0, 0)),
                  pl.BlockSpec((1, H2), lambda i: (0, 0)),
                  pl.BlockSpec((1, H2), lambda i: (0, 0))],
        out_specs=pl.BlockSpec((G, H2), lambda i: (0, 0)),
        out_shape=jax.ShapeDtypeStruct((G, H2), jnp.float32),
        scratch_shapes=[pltpu.VMEM((G, D), jnp.float32),
                        pltpu.VMEM((G, D), jnp.float32)],
    )(a, b2, bt, l1w, l1b, g1, be1, l2w, l2b, g2, be2)


def _edge_body(m_hbm, ei_hbm, z_hbm, out_hbm, acc_sp, src_i, dst_i, rows, sem):
    cid = lax.axis_index("c")
    sid = lax.axis_index("s")
    wid = cid * NS + sid
    r0 = pl.multiple_of(sid * ZR, 8)
    @pl.when(sid < NS - 1)
    def _():
        pltpu.sync_copy(z_hbm.at[pl.ds(0, ZR)], acc_sp.at[pl.ds(r0, ZR)])

    @pl.when(sid == NS - 1)
    def _():
        pltpu.sync_copy(z_hbm.at[pl.ds(0, ZL)], acc_sp.at[pl.ds(r0, ZL)])

    pltpu.sync_copy(ei_hbm.at[0, wid], src_i)
    pltpu.sync_copy(ei_hbm.at[1, wid], dst_i)
    plsc.subcore_barrier()

    def body(i, carry):
        pltpu.async_copy(m_hbm.at[src_i.at[i]], rows, sem).wait()
        pltpu.sync_copy(rows, acc_sp.at[dst_i.at[i]], add=True)
        return carry

    lax.fori_loop(0, NCH, body, 0)
    plsc.subcore_barrier()

    @pl.when(sid < NS - 1)
    def _():
        pltpu.sync_copy(acc_sp.at[pl.ds(r0, ZR)],
                        out_hbm.at[cid, pl.ds(r0, ZR)])

    @pl.when(sid == NS - 1)
    def _():
        pltpu.sync_copy(acc_sp.at[pl.ds(r0, ZL)],
                        out_hbm.at[cid, pl.ds(r0, ZL)])


def _edge_agg(m, ei_r, zeros):
    mesh = plsc.VectorSubcoreMesh(core_axis_name="c", subcore_axis_name="s")
    k = functools.partial(
        pl.kernel,
        mesh=mesh,
        out_type=jax.ShapeDtypeStruct((NC, N, D), jnp.float32),
        scratch_types=[
            pltpu.VMEM_SHARED((N, D), jnp.float32),
            pltpu.VMEM((NCH, CH), jnp.int32),
            pltpu.VMEM((NCH, CH), jnp.int32),
            pltpu.VMEM((CH, D), jnp.float32),
            pltpu.SemaphoreType.DMA,
        ],
    )(_edge_body)
    return k(m, ei_r, zeros)


def kernel(x, edge_index, batch, W1, b1, W2, b2,
           lin1_W, lin1_b, bn1_g, bn1_b, lin2_W, lin2_b, bn2_g, bn2_b):
    ei_r = edge_index.reshape(2, NW, NCH, CH)
    zeros = jnp.zeros((ZR, D), jnp.float32)
    bt = batch.reshape(NB, 1, BN)

    m1 = _mm(x, W1)
    a1 = _edge_agg(m1, ei_r, zeros)
    m2 = _layer2(a1, b1.reshape(1, D), W2)
    a2 = _edge_agg(m2, ei_r, zeros)
    return _head(a2, b2.reshape(1, D), bt,
                 lin1_W, lin1_b.reshape(1, -1),
                 bn1_g.reshape(1, -1), bn1_b.reshape(1, -1),
                 lin2_W, lin2_b.reshape(1, -1),
                 bn2_g.reshape(1, -1), bn2_b.reshape(1, -1))


# speedup vs baseline: 10.6135x; 1.4060x over previous
import functools

import jax
import jax.numpy as jnp
from jax import lax
from jax.experimental import pallas as pl
from jax.experimental.pallas import tpu as pltpu
from jax.experimental.pallas import tpu_sc as plsc

N = 10000
D = 128
E = 320000
G = 64

BN = 1000
NB = N // BN

NC = 2
NS = 16
NW = NC * NS
EPT = E // NW
CH = 80
NCH = EPT // CH
ZR = 632
ZL = N - (NS - 1) * ZR


def _mm_body(x_ref, w_ref, o_ref):
    o_ref[...] = jnp.dot(x_ref[...], w_ref[...],
                         preferred_element_type=jnp.float32)


def _mm(x, w):
    return pl.pallas_call(
        _mm_body,
        grid=(NB,),
        in_specs=[pl.BlockSpec((BN, D), lambda i: (i, 0)),
                  pl.BlockSpec((D, D), lambda i: (0, 0))],
        out_specs=pl.BlockSpec((BN, D), lambda i: (i, 0)),
        out_shape=jax.ShapeDtypeStruct((N, D), jnp.float32),
    )(x, w)


def _layer2_body(a_ref, b_ref, w_ref, o_ref):
    h = jnp.maximum(a_ref[0] + a_ref[1] + b_ref[...], 0.0)
    o_ref[...] = jnp.dot(h, w_ref[...], preferred_element_type=jnp.float32)


def _layer2(a, b, w):
    return pl.pallas_call(
        _layer2_body,
        grid=(NB,),
        in_specs=[pl.BlockSpec((NC, BN, D), lambda i: (0, i, 0)),
                  pl.BlockSpec((1, D), lambda i: (0, 0)),
                  pl.BlockSpec((D, D), lambda i: (0, 0))],
        out_specs=pl.BlockSpec((BN, D), lambda i: (i, 0)),
        out_shape=jax.ShapeDtypeStruct((N, D), jnp.float32),
    )(a, b, w)


def _head_body(a_ref, b2_ref, bt_ref, l1w, l1b, g1, be1, l2w, l2b, g2, be2,
               o_ref, sums, cnts):
    i = pl.program_id(0)

    @pl.when(i == 0)
    def _():
        sums[...] = jnp.zeros_like(sums)
        cnts[...] = jnp.zeros_like(cnts)

    h = jnp.maximum(a_ref[0] + a_ref[1] + b2_ref[...], 0.0)
    bt = bt_ref[0]
    gid = lax.broadcasted_iota(jnp.int32, (G, BN), 0)
    ohT = (gid == bt).astype(jnp.float32)
    dn = (((1,), (0,)), ((), ()))
    sums[...] += lax.dot_general(ohT, h, dn,
                                 preferred_element_type=jnp.float32, precision=lax.Precision.HIGHEST)
    cnts[...] += lax.dot_general(ohT, jnp.ones((BN, D), jnp.float32), dn,
                                 preferred_element_type=jnp.float32, precision=lax.Precision.HIGHEST)

    @pl.when(i == NB - 1)
    def _():
        pooled = sums[...] / jnp.maximum(cnts[...], 1.0)
        dnT = (((1,), (1,)), ((), ()))
        z = lax.dot_general(pooled, l1w[...], dnT,
                            preferred_element_type=jnp.float32) + l1b[...]
        z = jnp.maximum(z, 0.0)
        mu = jnp.mean(z, axis=0, keepdims=True)
        var = jnp.mean((z - mu) ** 2, axis=0, keepdims=True)
        z = (z - mu) * lax.rsqrt(var + 1e-5) * g1[...] + be1[...]
        z = lax.dot_general(z, l2w[...], dnT,
                            preferred_element_type=jnp.float32) + l2b[...]
        z = jnp.maximum(z, 0.0)
        mu = jnp.mean(z, axis=0, keepdims=True)
        var = jnp.mean((z - mu) ** 2, axis=0, keepdims=True)
        z = (z - mu) * lax.rsqrt(var + 1e-5) * g2[...] + be2[...]
        mx = jnp.max(z, axis=1, keepdims=True)
        z = z - mx
        o_ref[...] = z - jnp.log(jnp.sum(jnp.exp(z), axis=1, keepdims=True))


def _head(a, b2, bt, l1w, l1b, g1, be1, l2w, l2b, g2, be2):
    H1 = l1w.shape[0]
    H2 = l2w.shape[0]
    return pl.pallas_call(
        _head_body,
        grid=(NB,),
        in_specs=[pl.BlockSpec((NC, BN, D), lambda i: (0, i, 0)),
                  pl.BlockSpec((1, D), lambda i: (0, 0)),
                  pl.BlockSpec((1, 1, BN), lambda i: (i, 0, 0)),
                  pl.BlockSpec((H1, D), lambda i: (0, 0)),
                  pl.BlockSpec((1, H1), lambda i: (0, 0)),
                  pl.BlockSpec((1, H1), lambda i: (0, 0)),
                  pl.BlockSpec((1, H1), lambda i: (0, 0)),
                  pl.BlockSpec((H2, H1), lambda i: (0, 0)),
                  pl.BlockSpec((1, H2), lambda i: (0, 0)),
                  pl.BlockSpec((1, H2), lambda i: (0, 0)),
                  pl.BlockSpec((1, H2), lambda i: (0, 0))],
        out_specs=pl.BlockSpec((G, H2), lambda i: (0, 0)),
        out_shape=jax.ShapeDtypeStruct((G, H2), jnp.float32),
        scratch_shapes=[pltpu.VMEM((G, D), jnp.float32),
                        pltpu.VMEM((G, D), jnp.float32)],
    )(a, b2, bt, l1w, l1b, g1, be1, l2w, l2b, g2, be2)


def _edge_body(m_hbm, ei_hbm, z_hbm, out_hbm, acc_sp,
               sc0, sc1, dc0, dc1, ra, rb,
               ssi0, ssi1, sdi0, sdi1, sg0, sg1):
    cid = lax.axis_index("c")
    sid = lax.axis_index("s")
    wid = cid * NS + sid
    r0 = pl.multiple_of(sid * ZR, 8)
    @pl.when(sid < NS - 1)
    def _():
        pltpu.sync_copy(z_hbm.at[pl.ds(0, ZR)], acc_sp.at[pl.ds(r0, ZR)])

    @pl.when(sid == NS - 1)
    def _():
        pltpu.sync_copy(z_hbm.at[pl.ds(0, ZL)], acc_sp.at[pl.ds(r0, ZL)])

    sc = (sc0, sc1)
    dc = (dc0, dc1)
    rw = (ra, rb)
    ssi = (ssi0, ssi1)
    sdi = (sdi0, sdi1)
    sg = (sg0, sg1)

    def idx_load(i, p):
        pltpu.async_copy(ei_hbm.at[0, wid, i], sc[p], ssi[p])
        pltpu.async_copy(ei_hbm.at[1, wid, i], dc[p], sdi[p])

    def gather(i, p):
        pltpu.make_async_copy(ei_hbm.at[0, wid, i], sc[p], ssi[p]).wait()
        pltpu.async_copy(m_hbm.at[sc[p].at[0]], rw[p], sg[p])

    def scatter(i, p):
        pltpu.make_async_copy(m_hbm.at[sc[p].at[0]], rw[p], sg[p]).wait()
        pltpu.make_async_copy(ei_hbm.at[1, wid, i], dc[p], sdi[p]).wait()
        pltpu.sync_copy(rw[p], acc_sp.at[dc[p].at[0]], add=True)

    plsc.subcore_barrier()

    idx_load(0, 0)
    idx_load(1, 1)
    gather(0, 0)

    def body(i, carry):
        def phase(p):
            q = 1 - p

            @pl.when(i + 1 < NCH)
            def _():
                gather(i + 1, q)

            scatter(i, p)

            @pl.when(i + 2 < NCH)
            def _():
                idx_load(i + 2, p)

        @pl.when(i % 2 == 0)
        def _():
            phase(0)

        @pl.when(i % 2 == 1)
        def _():
            phase(1)

        return carry

    lax.fori_loop(0, NCH, body, 0)
    plsc.subcore_barrier()

    @pl.when(sid < NS - 1)
    def _():
        pltpu.sync_copy(acc_sp.at[pl.ds(r0, ZR)],
                        out_hbm.at[cid, pl.ds(r0, ZR)])

    @pl.when(sid == NS - 1)
    def _():
        pltpu.sync_copy(acc_sp.at[pl.ds(r0, ZL)],
                        out_hbm.at[cid, pl.ds(r0, ZL)])


def _edge_agg(m, ei_r, zeros):
    mesh = plsc.VectorSubcoreMesh(core_axis_name="c", subcore_axis_name="s")
    k = functools.partial(
        pl.kernel,
        mesh=mesh,
        out_type=jax.ShapeDtypeStruct((NC, N, D), jnp.float32),
        scratch_types=[
            pltpu.VMEM_SHARED((N, D), jnp.float32),
            pltpu.VMEM((1, CH), jnp.int32),
            pltpu.VMEM((1, CH), jnp.int32),
            pltpu.VMEM((1, CH), jnp.int32),
            pltpu.VMEM((1, CH), jnp.int32),
            pltpu.VMEM((CH, D), jnp.float32),
            pltpu.VMEM((CH, D), jnp.float32),
            pltpu.SemaphoreType.DMA,
            pltpu.SemaphoreType.DMA,
            pltpu.SemaphoreType.DMA,
            pltpu.SemaphoreType.DMA,
            pltpu.SemaphoreType.DMA,
            pltpu.SemaphoreType.DMA,
        ],
    )(_edge_body)
    return k(m, ei_r, zeros)


def kernel(x, edge_index, batch, W1, b1, W2, b2,
           lin1_W, lin1_b, bn1_g, bn1_b, lin2_W, lin2_b, bn2_g, bn2_b):
    ei_r = edge_index.reshape(2, NW, NCH, 1, CH)
    zeros = jnp.zeros((ZR, D), jnp.float32)
    bt = batch.reshape(NB, 1, BN)

    m1 = _mm(x, W1)
    a1 = _edge_agg(m1, ei_r, zeros)
    m2 = _layer2(a1, b1.reshape(1, D), W2)
    a2 = _edge_agg(m2, ei_r, zeros)
    return _head(a2, b2.reshape(1, D), bt,
                 lin1_W, lin1_b.reshape(1, -1),
                 bn1_g.reshape(1, -1), bn1_b.reshape(1, -1),
                 lin2_W, lin2_b.reshape(1, -1),
                 bn2_g.reshape(1, -1), bn2_b.reshape(1, -1))


# speedup vs baseline: 11.3745x; 1.0717x over previous
import functools

import jax
import jax.numpy as jnp
from jax import lax
from jax.experimental import pallas as pl
from jax.experimental.pallas import tpu as pltpu
from jax.experimental.pallas import tpu_sc as plsc

N = 10000
D = 128
E = 320000
G = 64

BN = 1000
NB = N // BN

NC = 2
NS = 16
NW = NC * NS
EPT = E // NW
CH = 80
NCH = EPT // CH
ZR = 632
ZL = N - (NS - 1) * ZR


def _mm_body(x_ref, w_ref, o_ref):
    o_ref[...] = jnp.dot(x_ref[...], w_ref[...],
                         preferred_element_type=jnp.float32)


def _mm(x, w):
    return pl.pallas_call(
        _mm_body,
        grid=(NB,),
        in_specs=[pl.BlockSpec((BN, D), lambda i: (i, 0)),
                  pl.BlockSpec((D, D), lambda i: (0, 0))],
        out_specs=pl.BlockSpec((BN, D), lambda i: (i, 0)),
        out_shape=jax.ShapeDtypeStruct((N, D), jnp.float32),
    )(x, w)


def _layer2_body(a_ref, b_ref, w_ref, o_ref):
    h = jnp.maximum(a_ref[0] + a_ref[1] + b_ref[...], 0.0)
    o_ref[...] = jnp.dot(h, w_ref[...], preferred_element_type=jnp.float32)


def _layer2(a, b, w):
    return pl.pallas_call(
        _layer2_body,
        grid=(NB,),
        in_specs=[pl.BlockSpec((NC, BN, D), lambda i: (0, i, 0)),
                  pl.BlockSpec((1, D), lambda i: (0, 0)),
                  pl.BlockSpec((D, D), lambda i: (0, 0))],
        out_specs=pl.BlockSpec((BN, D), lambda i: (i, 0)),
        out_shape=jax.ShapeDtypeStruct((N, D), jnp.float32),
    )(a, b, w)


def _head_body(a_ref, b2_ref, bt_ref, l1w, l1b, g1, be1, l2w, l2b, g2, be2,
               o_ref, sums, cnts):
    i = pl.program_id(0)

    @pl.when(i == 0)
    def _():
        sums[...] = jnp.zeros_like(sums)
        cnts[...] = jnp.zeros_like(cnts)

    h = jnp.maximum(a_ref[0] + a_ref[1] + b2_ref[...], 0.0)
    bt = bt_ref[0]
    gid = lax.broadcasted_iota(jnp.int32, (G, BN), 0)
    ohT = (gid == bt).astype(jnp.float32)
    dn = (((1,), (0,)), ((), ()))
    sums[...] += lax.dot_general(ohT, h, dn,
                                 preferred_element_type=jnp.float32, precision=lax.Precision.HIGHEST)
    cnts[...] += lax.dot_general(ohT, jnp.ones((BN, D), jnp.float32), dn,
                                 preferred_element_type=jnp.float32, precision=lax.Precision.HIGHEST)

    @pl.when(i == NB - 1)
    def _():
        pooled = sums[...] / jnp.maximum(cnts[...], 1.0)
        dnT = (((1,), (1,)), ((), ()))
        z = lax.dot_general(pooled, l1w[...], dnT,
                            preferred_element_type=jnp.float32) + l1b[...]
        z = jnp.maximum(z, 0.0)
        mu = jnp.mean(z, axis=0, keepdims=True)
        var = jnp.mean((z - mu) ** 2, axis=0, keepdims=True)
        z = (z - mu) * lax.rsqrt(var + 1e-5) * g1[...] + be1[...]
        z = lax.dot_general(z, l2w[...], dnT,
                            preferred_element_type=jnp.float32) + l2b[...]
        z = jnp.maximum(z, 0.0)
        mu = jnp.mean(z, axis=0, keepdims=True)
        var = jnp.mean((z - mu) ** 2, axis=0, keepdims=True)
        z = (z - mu) * lax.rsqrt(var + 1e-5) * g2[...] + be2[...]
        mx = jnp.max(z, axis=1, keepdims=True)
        z = z - mx
        o_ref[...] = z - jnp.log(jnp.sum(jnp.exp(z), axis=1, keepdims=True))


def _head(a, b2, bt, l1w, l1b, g1, be1, l2w, l2b, g2, be2):
    H1 = l1w.shape[0]
    H2 = l2w.shape[0]
    return pl.pallas_call(
        _head_body,
        grid=(NB,),
        in_specs=[pl.BlockSpec((NC, BN, D), lambda i: (0, i, 0)),
                  pl.BlockSpec((1, D), lambda i: (0, 0)),
                  pl.BlockSpec((1, 1, BN), lambda i: (i, 0, 0)),
                  pl.BlockSpec((H1, D), lambda i: (0, 0)),
                  pl.BlockSpec((1, H1), lambda i: (0, 0)),
                  pl.BlockSpec((1, H1), lambda i: (0, 0)),
                  pl.BlockSpec((1, H1), lambda i: (0, 0)),
                  pl.BlockSpec((H2, H1), lambda i: (0, 0)),
                  pl.BlockSpec((1, H2), lambda i: (0, 0)),
                  pl.BlockSpec((1, H2), lambda i: (0, 0)),
                  pl.BlockSpec((1, H2), lambda i: (0, 0))],
        out_specs=pl.BlockSpec((G, H2), lambda i: (0, 0)),
        out_shape=jax.ShapeDtypeStruct((G, H2), jnp.float32),
        scratch_shapes=[pltpu.VMEM((G, D), jnp.float32),
                        pltpu.VMEM((G, D), jnp.float32)],
    )(a, b2, bt, l1w, l1b, g1, be1, l2w, l2b, g2, be2)


def _edge_body(m_hbm, ei_hbm, z_hbm, out_hbm, acc_sp,
               sc0, sc1, dc0, dc1, dc2, ra, rb, rc,
               ssi0, ssi1, sdi0, sdi1, sdi2, sg0, sg1, sg2,
               ssc0, ssc1, ssc2):
    cid = lax.axis_index("c")
    sid = lax.axis_index("s")
    wid = cid * NS + sid
    r0 = pl.multiple_of(sid * ZR, 8)
    @pl.when(sid < NS - 1)
    def _():
        pltpu.sync_copy(z_hbm.at[pl.ds(0, ZR)], acc_sp.at[pl.ds(r0, ZR)])

    @pl.when(sid == NS - 1)
    def _():
        pltpu.sync_copy(z_hbm.at[pl.ds(0, ZL)], acc_sp.at[pl.ds(r0, ZL)])

    sc = (sc0, sc1)
    dc = (dc0, dc1, dc2)
    rw = (ra, rb, rc)
    ssi = (ssi0, ssi1)
    sdi = (sdi0, sdi1, sdi2)
    sg = (sg0, sg1, sg2)
    ssc = (ssc0, ssc1, ssc2)

    def load_src(i, p2):
        pltpu.async_copy(ei_hbm.at[0, wid, i], sc[p2], ssi[p2])

    def load_dst(i, p3):
        pltpu.async_copy(ei_hbm.at[1, wid, i], dc[p3], sdi[p3])

    def wait_src(i, p2):
        pltpu.make_async_copy(ei_hbm.at[0, wid, i], sc[p2], ssi[p2]).wait()

    def wait_dst(i, p3):
        pltpu.make_async_copy(ei_hbm.at[1, wid, i], dc[p3], sdi[p3]).wait()

    def gather(i, p2, p3):
        wait_src(i, p2)
        pltpu.async_copy(m_hbm.at[sc[p2].at[0]], rw[p3], sg[p3])

    def wait_gather(p2, p3):
        pltpu.make_async_copy(m_hbm.at[sc[p2].at[0]], rw[p3], sg[p3]).wait()

    def scatter(i, p3):
        wait_dst(i, p3)
        pltpu.async_copy(rw[p3], acc_sp.at[dc[p3].at[0]], ssc[p3], add=True)

    def wait_scatter(p3):
        pltpu.make_async_copy(rw[p3], acc_sp.at[dc[p3].at[0]], ssc[p3]).wait()

    plsc.subcore_barrier()

    load_src(0, 0)
    load_src(1, 1)
    load_dst(0, 0)
    load_dst(1, 1)
    load_dst(2, 2)
    gather(0, 0, 0)

    def phase(i, p2, p3):
        q2 = 1 - p2
        q3 = (p3 + 1) % 3

        @pl.when(i + 1 < NCH)
        def _():
            @pl.when(i >= 2)
            def _():
                wait_scatter(q3)

            @pl.when(i + 1 >= 3)
            def _():
                load_dst(i + 1, q3)

            gather(i + 1, q2, q3)

        wait_gather(p2, p3)
        scatter(i, p3)

        @pl.when(i + 2 < NCH)
        def _():
            load_src(i + 2, p2)

    def body(i, carry):
        for k in range(6):
            @pl.when(i % 6 == k)
            def _(k=k):
                phase(i, k % 2, k % 3)
        return carry

    lax.fori_loop(0, NCH, body, 0)
    wait_scatter((NCH - 2) % 3)
    wait_scatter((NCH - 1) % 3)
    plsc.subcore_barrier()

    @pl.when(sid < NS - 1)
    def _():
        pltpu.sync_copy(acc_sp.at[pl.ds(r0, ZR)],
                        out_hbm.at[cid, pl.ds(r0, ZR)])

    @pl.when(sid == NS - 1)
    def _():
        pltpu.sync_copy(acc_sp.at[pl.ds(r0, ZL)],
                        out_hbm.at[cid, pl.ds(r0, ZL)])


def _edge_agg(m, ei_r, zeros):
    mesh = plsc.VectorSubcoreMesh(core_axis_name="c", subcore_axis_name="s")
    k = functools.partial(
        pl.kernel,
        mesh=mesh,
        out_type=jax.ShapeDtypeStruct((NC, N, D), jnp.float32),
        scratch_types=[
            pltpu.VMEM_SHARED((N, D), jnp.float32),
            pltpu.VMEM((1, CH), jnp.int32),
            pltpu.VMEM((1, CH), jnp.int32),
            pltpu.VMEM((1, CH), jnp.int32),
            pltpu.VMEM((1, CH), jnp.int32),
            pltpu.VMEM((1, CH), jnp.int32),
            pltpu.VMEM((CH, D), jnp.float32),
            pltpu.VMEM((CH, D), jnp.float32),
            pltpu.VMEM((CH, D), jnp.float32),
        ] + [pltpu.SemaphoreType.DMA] * 11,
    )(_edge_body)
    return k(m, ei_r, zeros)


def kernel(x, edge_index, batch, W1, b1, W2, b2,
           lin1_W, lin1_b, bn1_g, bn1_b, lin2_W, lin2_b, bn2_g, bn2_b):
    ei_r = edge_index.reshape(2, NW, NCH, 1, CH)
    zeros = jnp.zeros((ZR, D), jnp.float32)
    bt = batch.reshape(NB, 1, BN)

    m1 = _mm(x, W1)
    a1 = _edge_agg(m1, ei_r, zeros)
    m2 = _layer2(a1, b1.reshape(1, D), W2)
    a2 = _edge_agg(m2, ei_r, zeros)
    return _head(a2, b2.reshape(1, D), bt,
                 lin1_W, lin1_b.reshape(1, -1),
                 bn1_g.reshape(1, -1), bn1_b.reshape(1, -1),
                 lin2_W, lin2_b.reshape(1, -1),
                 bn2_g.reshape(1, -1), bn2_b.reshape(1, -1))


# speedup vs baseline: 11.4357x; 1.0054x over previous
import functools

import jax
import jax.numpy as jnp
from jax import lax
from jax.experimental import pallas as pl
from jax.experimental.pallas import tpu as pltpu
from jax.experimental.pallas import tpu_sc as plsc

N = 10000
D = 128
E = 320000
G = 64

BN = 1000
NB = N // BN

NC = 2
NS = 16
NW = NC * NS
EPT = E // NW
CH = 80
NCH = EPT // CH
ZR = 632
ZL = N - (NS - 1) * ZR


def _mm_body(x_ref, w_ref, o_ref):
    o_ref[...] = jnp.dot(x_ref[...], w_ref[...],
                         preferred_element_type=jnp.float32)


def _mm(x, w):
    return pl.pallas_call(
        _mm_body,
        grid=(NB,),
        in_specs=[pl.BlockSpec((BN, D), lambda i: (i, 0)),
                  pl.BlockSpec((D, D), lambda i: (0, 0))],
        out_specs=pl.BlockSpec((BN, D), lambda i: (i, 0)),
        out_shape=jax.ShapeDtypeStruct((N, D), jnp.float32),
    )(x, w)


def _layer2_body(a_ref, b_ref, w_ref, o_ref):
    h = jnp.maximum(a_ref[0] + a_ref[1] + b_ref[...], 0.0)
    o_ref[...] = jnp.dot(h, w_ref[...], preferred_element_type=jnp.float32)


def _layer2(a, b, w):
    return pl.pallas_call(
        _layer2_body,
        grid=(NB,),
        in_specs=[pl.BlockSpec((NC, BN, D), lambda i: (0, i, 0)),
                  pl.BlockSpec((1, D), lambda i: (0, 0)),
                  pl.BlockSpec((D, D), lambda i: (0, 0))],
        out_specs=pl.BlockSpec((BN, D), lambda i: (i, 0)),
        out_shape=jax.ShapeDtypeStruct((N, D), jnp.float32),
    )(a, b, w)


def _head_body(a_ref, b2_ref, bt_ref, l1w, l1b, g1, be1, l2w, l2b, g2, be2,
               o_ref, sums, cnts):
    i = pl.program_id(0)

    @pl.when(i == 0)
    def _():
        sums[...] = jnp.zeros_like(sums)
        cnts[...] = jnp.zeros_like(cnts)

    h = jnp.maximum(a_ref[0] + a_ref[1] + b2_ref[...], 0.0)
    bt = bt_ref[0]
    gid = lax.broadcasted_iota(jnp.int32, (G, BN), 0)
    ohT = (gid == bt).astype(jnp.float32)
    dn = (((1,), (0,)), ((), ()))
    sums[...] += lax.dot_general(ohT, h, dn,
                                 preferred_element_type=jnp.float32, precision=lax.Precision.HIGHEST)
    cnts[...] += lax.dot_general(ohT, jnp.ones((BN, D), jnp.float32), dn,
                                 preferred_element_type=jnp.float32, precision=lax.Precision.HIGHEST)

    @pl.when(i == NB - 1)
    def _():
        pooled = sums[...] / jnp.maximum(cnts[...], 1.0)
        dnT = (((1,), (1,)), ((), ()))
        z = lax.dot_general(pooled, l1w[...], dnT,
                            preferred_element_type=jnp.float32) + l1b[...]
        z = jnp.maximum(z, 0.0)
        mu = jnp.mean(z, axis=0, keepdims=True)
        var = jnp.mean((z - mu) ** 2, axis=0, keepdims=True)
        z = (z - mu) * lax.rsqrt(var + 1e-5) * g1[...] + be1[...]
        z = lax.dot_general(z, l2w[...], dnT,
                            preferred_element_type=jnp.float32) + l2b[...]
        z = jnp.maximum(z, 0.0)
        mu = jnp.mean(z, axis=0, keepdims=True)
        var = jnp.mean((z - mu) ** 2, axis=0, keepdims=True)
        z = (z - mu) * lax.rsqrt(var + 1e-5) * g2[...] + be2[...]
        mx = jnp.max(z, axis=1, keepdims=True)
        z = z - mx
        o_ref[...] = z - jnp.log(jnp.sum(jnp.exp(z), axis=1, keepdims=True))


def _head(a, b2, bt, l1w, l1b, g1, be1, l2w, l2b, g2, be2):
    H1 = l1w.shape[0]
    H2 = l2w.shape[0]
    return pl.pallas_call(
        _head_body,
        grid=(NB,),
        in_specs=[pl.BlockSpec((NC, BN, D), lambda i: (0, i, 0)),
                  pl.BlockSpec((1, D), lambda i: (0, 0)),
                  pl.BlockSpec((1, 1, BN), lambda i: (i, 0, 0)),
                  pl.BlockSpec((H1, D), lambda i: (0, 0)),
                  pl.BlockSpec((1, H1), lambda i: (0, 0)),
                  pl.BlockSpec((1, H1), lambda i: (0, 0)),
                  pl.BlockSpec((1, H1), lambda i: (0, 0)),
                  pl.BlockSpec((H2, H1), lambda i: (0, 0)),
                  pl.BlockSpec((1, H2), lambda i: (0, 0)),
                  pl.BlockSpec((1, H2), lambda i: (0, 0)),
                  pl.BlockSpec((1, H2), lambda i: (0, 0))],
        out_specs=pl.BlockSpec((G, H2), lambda i: (0, 0)),
        out_shape=jax.ShapeDtypeStruct((G, H2), jnp.float32),
        scratch_shapes=[pltpu.VMEM((G, D), jnp.float32),
                        pltpu.VMEM((G, D), jnp.float32)],
    )(a, b2, bt, l1w, l1b, g1, be1, l2w, l2b, g2, be2)


def _edge_body(m_hbm, ei_hbm, z_hbm, out_hbm, acc_sp,
               sc0, sc1, dc0, dc1, dc2, dc3, ra, rb, rc, rd,
               ssi0, ssi1, sdi0, sdi1, sdi2, sdi3, sg0, sg1, sg2, sg3,
               ssc0, ssc1, ssc2, ssc3):
    cid = lax.axis_index("c")
    sid = lax.axis_index("s")
    wid = cid * NS + sid
    r0 = pl.multiple_of(sid * ZR, 8)

    sc = (sc0, sc1)
    dc = (dc0, dc1, dc2, dc3)
    rw = (ra, rb, rc, rd)
    ssi = (ssi0, ssi1)
    sdi = (sdi0, sdi1, sdi2, sdi3)
    sg = (sg0, sg1, sg2, sg3)
    ssc = (ssc0, ssc1, ssc2, ssc3)

    def load_src(i, p2):
        pltpu.async_copy(ei_hbm.at[0, wid, i], sc[p2], ssi[p2])

    def load_dst(i, p4):
        pltpu.async_copy(ei_hbm.at[1, wid, i], dc[p4], sdi[p4])

    def wait_src(i, p2):
        pltpu.make_async_copy(ei_hbm.at[0, wid, i], sc[p2], ssi[p2]).wait()

    def wait_dst(i, p4):
        pltpu.make_async_copy(ei_hbm.at[1, wid, i], dc[p4], sdi[p4]).wait()

    def gather(i, p2, p4):
        wait_src(i, p2)
        pltpu.async_copy(m_hbm.at[sc[p2].at[0]], rw[p4], sg[p4])

    def wait_gather(p2, p4):
        pltpu.make_async_copy(m_hbm.at[sc[p2].at[0]], rw[p4], sg[p4]).wait()

    def scatter(i, p4):
        wait_dst(i, p4)
        pltpu.async_copy(rw[p4], acc_sp.at[dc[p4].at[0]], ssc[p4], add=True)

    def wait_scatter(p4):
        pltpu.make_async_copy(rw[p4], acc_sp.at[dc[p4].at[0]], ssc[p4]).wait()

    load_src(0, 0)
    load_src(1, 1)
    load_dst(0, 0)
    load_dst(1, 1)
    load_dst(2, 2)
    load_dst(3, 3)
    gather(0, 0, 0)

    @pl.when(sid < NS - 1)
    def _():
        pltpu.sync_copy(z_hbm.at[pl.ds(0, ZR)], acc_sp.at[pl.ds(r0, ZR)])

    @pl.when(sid == NS - 1)
    def _():
        pltpu.sync_copy(z_hbm.at[pl.ds(0, ZL)], acc_sp.at[pl.ds(r0, ZL)])

    plsc.subcore_barrier()

    def phase(i, p4):
        p2 = p4 % 2
        q2 = 1 - p2
        q4 = (p4 + 1) % 4

        @pl.when(i + 1 < NCH)
        def _():
            @pl.when(i >= 3)
            def _():
                wait_scatter(q4)

            @pl.when(i + 1 >= 4)
            def _():
                load_dst(i + 1, q4)

            gather(i + 1, q2, q4)

        wait_gather(p2, p4)
        scatter(i, p4)

        @pl.when(i + 2 < NCH)
        def _():
            load_src(i + 2, p2)

    def body(i, carry):
        for k in range(4):
            @pl.when(i % 4 == k)
            def _(k=k):
                phase(i, k)
        return carry

    lax.fori_loop(0, NCH, body, 0)
    wait_scatter((NCH - 3) % 4)
    wait_scatter((NCH - 2) % 4)
    wait_scatter((NCH - 1) % 4)
    plsc.subcore_barrier()

    @pl.when(sid < NS - 1)
    def _():
        pltpu.sync_copy(acc_sp.at[pl.ds(r0, ZR)],
                        out_hbm.at[cid, pl.ds(r0, ZR)])

    @pl.when(sid == NS - 1)
    def _():
        pltpu.sync_copy(acc_sp.at[pl.ds(r0, ZL)],
                        out_hbm.at[cid, pl.ds(r0, ZL)])


def _edge_agg(m, ei_r, zeros):
    mesh = plsc.VectorSubcoreMesh(core_axis_name="c", subcore_axis_name="s")
    k = functools.partial(
        pl.kernel,
        mesh=mesh,
        out_type=jax.ShapeDtypeStruct((NC, N, D), jnp.float32),
        scratch_types=[
            pltpu.VMEM_SHARED((N, D), jnp.float32),
            pltpu.VMEM((1, CH), jnp.int32),
            pltpu.VMEM((1, CH), jnp.int32),
            pltpu.VMEM((1, CH), jnp.int32),
            pltpu.VMEM((1, CH), jnp.int32),
            pltpu.VMEM((1, CH), jnp.int32),
            pltpu.VMEM((1, CH), jnp.int32),
            pltpu.VMEM((CH, D), jnp.float32),
            pltpu.VMEM((CH, D), jnp.float32),
            pltpu.VMEM((CH, D), jnp.float32),
            pltpu.VMEM((CH, D), jnp.float32),
        ] + [pltpu.SemaphoreType.DMA] * 14,
    )(_edge_body)
    return k(m, ei_r, zeros)


def kernel(x, edge_index, batch, W1, b1, W2, b2,
           lin1_W, lin1_b, bn1_g, bn1_b, lin2_W, lin2_b, bn2_g, bn2_b):
    ei_r = edge_index.reshape(2, NW, NCH, 1, CH)
    zeros = jnp.zeros((ZR, D), jnp.float32)
    bt = batch.reshape(NB, 1, BN)

    m1 = _mm(x, W1)
    a1 = _edge_agg(m1, ei_r, zeros)
    m2 = _layer2(a1, b1.reshape(1, D), W2)
    a2 = _edge_agg(m2, ei_r, zeros)
    return _head(a2, b2.reshape(1, D), bt,
                 lin1_W, lin1_b.reshape(1, -1),
                 bn1_g.reshape(1, -1), bn1_b.reshape(1, -1),
                 lin2_W, lin2_b.reshape(1, -1),
                 bn2_g.reshape(1, -1), bn2_b.reshape(1, -1))
